# baseline (speedup 1.0000x reference)
import jax
import jax.numpy as jnp
from jax.experimental import pallas as pl
from jax.experimental.pallas import tpu as pltpu

_N = 4096
_DP = 128
_D = 64
_RB = 1024
_NT = 6
_SRC = (0, 1, 0, 2, 0, 1)
_DST = (1, 2, 2, 3, 3, 3)
_FIRST = (1, 1, 0, 1, 0, 0)


def _cell_kernel(aidx_ref, x_ref, w_ref, b_ref, g_ref, bt_ref, adj_l_ref,
                 adj_r_ref, o_ref, states_ref):
    t = pl.program_id(0)
    rb = pl.program_id(1)

    @pl.when(jnp.logical_and(t == 0, rb == 0))
    def _():
        h = jax.lax.dot_general(x_ref[...], w_ref[...],
                                (((1,), (1,)), ((), ())),
                                preferred_element_type=jnp.float32)
        states_ref[0] = h + b_ref[0][None, :]

    a_l = adj_l_ref[0].astype(jnp.bfloat16)
    a_r = adj_r_ref[0].astype(jnp.bfloat16)
    row = pl.ds(rb * _RB, _RB)
    for tt in range(_NT):
        @pl.when(t == tt)
        def _(tt=tt):
            rhs = states_ref[_SRC[tt]].astype(jnp.bfloat16)
            contrib = jnp.dot(a_l, rhs[:_N // 2],
                              preferred_element_type=jnp.float32) \
                + jnp.dot(a_r, rhs[_N // 2:],
                          preferred_element_type=jnp.float32)
            if _FIRST[tt]:
                states_ref[_DST[tt], row] = contrib
            else:
                states_ref[_DST[tt], row] += contrib

    @pl.when(t == _NT - 1)
    def _():
        s = states_ref[3, row]
        mu = jnp.mean(s, axis=-1, keepdims=True)
        var = jnp.mean((s - mu) ** 2, axis=-1, keepdims=True)
        ln = (s - mu) * jax.lax.rsqrt(var + 1e-5) * g_ref[0][None, :] \
            + bt_ref[0][None, :]
        o_ref[...] = 0.5 * ln * (1.0 + jax.lax.erf(ln * 0.7071067811865476))


def kernel(x, adjs, idxes_seq, idxes_res, W, b, gamma, beta):
    iseq = idxes_seq.astype(jnp.int32)
    ires = idxes_res.astype(jnp.int32)
    aidx = jnp.stack([iseq[0], iseq[1], ires[0], iseq[2], ires[1], ires[2]])

    grid_spec = pltpu.PrefetchScalarGridSpec(
        num_scalar_prefetch=1,
        grid=(_NT, _N // _RB),
        in_specs=[
            pl.BlockSpec((_N, _DP), lambda t, rb, a: (0, 0)),
            pl.BlockSpec((_D, _DP), lambda t, rb, a: (0, 0)),
            pl.BlockSpec((1, _D), lambda t, rb, a: (0, 0)),
            pl.BlockSpec((1, _D), lambda t, rb, a: (0, 0)),
            pl.BlockSpec((1, _D), lambda t, rb, a: (0, 0)),
            pl.BlockSpec((1, _RB, _N // 2), lambda t, rb, a: (a[t], rb, 0)),
            pl.BlockSpec((1, _RB, _N // 2), lambda t, rb, a: (a[t], rb, 1)),
        ],
        out_specs=pl.BlockSpec((_RB, _D), lambda t, rb, a: (rb, 0)),
        scratch_shapes=[pltpu.VMEM((4, _N, _D), jnp.float32)],
    )
    return pl.pallas_call(
        _cell_kernel,
        grid_spec=grid_spec,
        out_shape=jax.ShapeDtypeStruct((_N, _D), jnp.float32),
        compiler_params=pltpu.CompilerParams(
            vmem_limit_bytes=100 * 1024 * 1024),
    )(aidx, x, W, b.reshape(1, _D), gamma.reshape(1, _D),
      beta.reshape(1, _D), adjs, adjs)


# speedup vs baseline: 1.0769x; 1.0769x over previous
import jax
import jax.numpy as jnp
from jax.experimental import pallas as pl
from jax.experimental.pallas import tpu as pltpu

_N = 4096
_DP = 128
_D = 64
_RB = 1024
_NT = 6
_SRC = (0, 1, 0, 2, 0, 1)
_DST = (1, 2, 2, 3, 3, 3)
_FIRST = (1, 1, 0, 1, 0, 0)


def _term_adj_index(t, iseq, ires):
    return jnp.where(
        t == 0, iseq[0],
        jnp.where(t == 1, iseq[1],
                  jnp.where(t == 2, ires[0],
                            jnp.where(t == 3, iseq[2],
                                      jnp.where(t == 4, ires[1], ires[2])))))


def _cell_kernel(iseq_ref, ires_ref, x_ref, w_ref, b_ref, g_ref, bt_ref,
                 adj_ref, o_ref, states_ref):
    t = pl.program_id(0)
    rb = pl.program_id(1)

    @pl.when(jnp.logical_and(t == 0, rb == 0))
    def _():
        h = jax.lax.dot_general(x_ref[...], w_ref[...],
                                (((1,), (1,)), ((), ())),
                                preferred_element_type=jnp.float32)
        states_ref[0] = h + b_ref[0][None, :]

    a = adj_ref[0].astype(jnp.bfloat16)
    row = pl.ds(rb * _RB, _RB)
    for tt in range(_NT):
        @pl.when(t == tt)
        def _(tt=tt):
            contrib = jnp.dot(a, states_ref[_SRC[tt]].astype(jnp.bfloat16),
                              preferred_element_type=jnp.float32)
            if _FIRST[tt]:
                states_ref[_DST[tt], row] = contrib
            else:
                states_ref[_DST[tt], row] += contrib

    @pl.when(t == _NT - 1)
    def _():
        s = states_ref[3, row]
        mu = jnp.mean(s, axis=-1, keepdims=True)
        var = jnp.mean((s - mu) ** 2, axis=-1, keepdims=True)
        ln = (s - mu) * jax.lax.rsqrt(var + 1e-5) * g_ref[0][None, :] \
            + bt_ref[0][None, :]
        o_ref[...] = 0.5 * ln * (1.0 + jax.lax.erf(ln * 0.7071067811865476))


def kernel(x, adjs, idxes_seq, idxes_res, W, b, gamma, beta):
    iseq = idxes_seq.astype(jnp.int32)
    ires = idxes_res.astype(jnp.int32)

    grid_spec = pltpu.PrefetchScalarGridSpec(
        num_scalar_prefetch=2,
        grid=(_NT, _N // _RB),
        in_specs=[
            pl.BlockSpec((_N, _DP), lambda t, rb, s, r: (0, 0)),
            pl.BlockSpec((_D, _DP), lambda t, rb, s, r: (0, 0)),
            pl.BlockSpec((1, _D), lambda t, rb, s, r: (0, 0)),
            pl.BlockSpec((1, _D), lambda t, rb, s, r: (0, 0)),
            pl.BlockSpec((1, _D), lambda t, rb, s, r: (0, 0)),
            pl.BlockSpec((1, _RB, _N),
                         lambda t, rb, s, r: (_term_adj_index(t, s, r), rb, 0)),
        ],
        out_specs=pl.BlockSpec(
            (_RB, _D),
            lambda t, rb, s, r: (jnp.where(t == _NT - 1, rb, 0), 0)),
        scratch_shapes=[pltpu.VMEM((4, _N, _D), jnp.float32)],
    )
    return pl.pallas_call(
        _cell_kernel,
        grid_spec=grid_spec,
        out_shape=jax.ShapeDtypeStruct((_N, _D), jnp.float32),
        compiler_params=pltpu.CompilerParams(
            vmem_limit_bytes=100 * 1024 * 1024),
    )(iseq, ires, x, W, b.reshape(1, _D), gamma.reshape(1, _D),
      beta.reshape(1, _D), adjs)
